# Initial kernel scaffold; baseline (speedup 1.0000x reference)
import jax
import jax.numpy as jnp
from jax.experimental import pallas as pl


def kernel(sparse_adj_t, sparse_adj_f, class_emb, stu_embs, exer_embs, W1, b1, W2, b2, W3, b3, W4, b4, W5, b5, W6, b6):
    raise NotImplementedError("write your pallas kernel here")


# speedup vs baseline: 1.7703x; 1.7703x over previous
import numpy as np
import jax
import jax.numpy as jnp
from jax.experimental import pallas as pl

N_STU = 8192
N_EXER = 4096
D = 128
BM = 512
BN = 512

_INV_SQRT_D = float(1.0 / np.sqrt(D))


def _row_entropy(x):
    m = jnp.max(x, axis=1, keepdims=True)
    e = jnp.exp(x - m)
    p = e / jnp.sum(e, axis=1, keepdims=True)
    return -jnp.sum(p * jnp.log2(p + 1e-10), axis=1, keepdims=True) * _INV_SQRT_D


def _stu_kernel(adj_t_ref, adj_f_ref, stu_ref, exer_ref, class_ref,
                w1_ref, b1_ref, w2_ref, b2_ref, w3_ref, b3_ref, w4_ref, b4_ref,
                stu_new_ref, acc_t_ref, acc_f_ref, stu_sum_ref):
    i = pl.program_id(0)
    at = adj_t_ref[...]
    af = adj_f_ref[...]
    st = stu_ref[...]
    ex = exer_ref[...]

    @pl.when(i == 0)
    def _init():
        acc_t_ref[...] = jnp.zeros_like(acc_t_ref)
        acc_f_ref[...] = jnp.zeros_like(acc_f_ref)
        stu_sum_ref[...] = jnp.zeros_like(stu_sum_ref)

    dn = (((0,), (0,)), ((), ()))
    acc_t_ref[...] += jax.lax.dot_general(at, st, dn, preferred_element_type=jnp.float32)
    acc_f_ref[...] += jax.lax.dot_general(af, st, dn, preferred_element_type=jnp.float32)
    stu_sum_ref[...] += jnp.sum(st, axis=0, keepdims=True)

    ate = jnp.dot(at, ex, preferred_element_type=jnp.float32)
    afe = jnp.dot(af, ex, preferred_element_type=jnp.float32)

    w1, b1 = w1_ref[...], b1_ref[...]
    w2, b2 = w2_ref[...], b2_ref[...]
    w3, b3 = w3_ref[...], b3_ref[...]
    w4, b4 = w4_ref[...], b4_ref[...]

    stu_t = jnp.dot(ate, w1.T, preferred_element_type=jnp.float32) + b1
    stu_f = jnp.dot(afe, w2.T, preferred_element_type=jnp.float32) + b2
    s2s = jnp.dot(st, w4.T, preferred_element_type=jnp.float32) + b4
    c2s = (jnp.dot(class_ref[...], w3.T, preferred_element_type=jnp.float32) + b3) * (1.0 / N_STU)

    ent_t = _row_entropy(stu_t)
    ent_f = _row_entropy(stu_f)
    ent_c = _row_entropy(c2s)
    total = ent_t + ent_f + ent_c

    stu_new_ref[...] = (stu_t * ent_t + stu_f * ent_f + c2s * ent_c) / total / 3.0 + s2s


def _exer_kernel(acc_t_ref, acc_f_ref, exer_ref, class_ref, stu_sum_ref,
                 w1_ref, b1_ref, w2_ref, b2_ref, w3_ref, b3_ref,
                 w5_ref, b5_ref, w6_ref, b6_ref,
                 exer_new_ref, class_new_ref):
    w1, b1 = w1_ref[...], b1_ref[...]
    w2, b2 = w2_ref[...], b2_ref[...]

    et = jnp.dot(acc_t_ref[...], w1.T, preferred_element_type=jnp.float32) + b1
    ef = jnp.dot(acc_f_ref[...], w2.T, preferred_element_type=jnp.float32) + b2
    ent_t = _row_entropy(et)
    ent_f = _row_entropy(ef)
    total = ent_t + ent_f
    e2e = jnp.dot(exer_ref[...], w6_ref[...].T, preferred_element_type=jnp.float32) + b6_ref[...]
    exer_new_ref[...] = (et * ent_t + ef * ent_f) / total / 2.0 + e2e

    s2c = jnp.dot(stu_sum_ref[...] * (1.0 / N_STU), w3_ref[...].T,
                  preferred_element_type=jnp.float32) + b3_ref[...]
    c2c = jnp.dot(class_ref[...], w5_ref[...].T, preferred_element_type=jnp.float32) + b5_ref[...]
    class_new_ref[...] = s2c + c2c


def kernel(sparse_adj_t, sparse_adj_f, class_emb, stu_embs, exer_embs,
           W1, b1, W2, b2, W3, b3, W4, b4, W5, b5, W6, b6):
    f32 = jnp.float32
    b1r, b2r, b3r, b4r, b5r, b6r = (b.reshape(1, D) for b in (b1, b2, b3, b4, b5, b6))

    full = lambda shape: pl.BlockSpec(shape, lambda i: (0, 0))
    row_blk = lambda w: pl.BlockSpec((BM, w), lambda i: (i, 0))
    wspec = full((D, D))
    bspec = full((1, D))

    stu_new, acc_t, acc_f, stu_sum = pl.pallas_call(
        _stu_kernel,
        grid=(N_STU // BM,),
        in_specs=[
            row_blk(N_EXER), row_blk(N_EXER), row_blk(D),
            full((N_EXER, D)), full((1, D)),
            wspec, bspec, wspec, bspec, wspec, bspec, wspec, bspec,
        ],
        out_specs=[
            pl.BlockSpec((BM, D), lambda i: (i, 0)),
            full((N_EXER, D)), full((N_EXER, D)), full((1, D)),
        ],
        out_shape=[
            jax.ShapeDtypeStruct((N_STU, D), f32),
            jax.ShapeDtypeStruct((N_EXER, D), f32),
            jax.ShapeDtypeStruct((N_EXER, D), f32),
            jax.ShapeDtypeStruct((1, D), f32),
        ],
    )(sparse_adj_t, sparse_adj_f, stu_embs, exer_embs, class_emb,
      W1, b1r, W2, b2r, W3, b3r, W4, b4r)

    exer_row = lambda: pl.BlockSpec((BN, D), lambda j: (j, 0))
    exer_new, class_new = pl.pallas_call(
        _exer_kernel,
        grid=(N_EXER // BN,),
        in_specs=[
            exer_row(), exer_row(), exer_row(),
            full((1, D)), full((1, D)),
            wspec, bspec, wspec, bspec, wspec, bspec,
            wspec, bspec, wspec, bspec,
        ],
        out_specs=[
            pl.BlockSpec((BN, D), lambda j: (j, 0)),
            full((1, D)),
        ],
        out_shape=[
            jax.ShapeDtypeStruct((N_EXER, D), f32),
            jax.ShapeDtypeStruct((1, D), f32),
        ],
    )(acc_t, acc_f, exer_embs, class_emb, stu_sum,
      W1, b1r, W2, b2r, W3, b3r, W5, b5r, W6, b6r)

    return (class_new, stu_new, exer_new)
